# baseline (speedup 1.0000x reference)
import functools

import jax
import jax.numpy as jnp
from jax.experimental import pallas as pl
from jax.experimental.pallas import tpu as pltpu

B, S, D, E, TOP_K = 4, 4096, 4096, 64, 8
BT = 512


def _router_body(x_ref, m_ref, w_ref, b_ref, ew_ref, ei_ref):
    logits = jnp.dot(x_ref[...], w_ref[...], preferred_element_type=jnp.float32)
    logits = logits + b_ref[...]
    mask = m_ref[...]
    logits = jnp.where(mask != 1, -jnp.inf, logits)

    iota = jax.lax.broadcasted_iota(jnp.int32, (BT, E), 1)
    vals = logits
    top_vals = []
    top_idx = []
    for _ in range(TOP_K):
        m = jnp.max(vals, axis=-1, keepdims=True)
        idx = jnp.min(jnp.where(vals == m, iota, E), axis=-1, keepdims=True)
        top_vals.append(m)
        top_idx.append(idx)
        vals = jnp.where(iota == idx, -jnp.inf, vals)

    tv = jnp.concatenate(top_vals, axis=-1)
    ti = jnp.concatenate(top_idx, axis=-1)
    e = jnp.exp(tv - tv[:, 0:1])
    ew_ref[...] = e / jnp.sum(e, axis=-1, keepdims=True)
    ei_ref[...] = ti


@functools.partial(jax.jit, static_argnames=())
def kernel(x, attention_mask, W, b):
    T = B * S
    x2 = x.reshape(T, D)
    m2 = attention_mask.reshape(T, 1)
    wt = W.T
    b2 = b.reshape(1, E)

    grid = (T // BT,)
    ew, ei = pl.pallas_call(
        _router_body,
        grid=grid,
        in_specs=[
            pl.BlockSpec((BT, D), lambda i: (i, 0)),
            pl.BlockSpec((BT, 1), lambda i: (i, 0)),
            pl.BlockSpec((D, E), lambda i: (0, 0)),
            pl.BlockSpec((1, E), lambda i: (0, 0)),
        ],
        out_specs=[
            pl.BlockSpec((BT, TOP_K), lambda i: (i, 0)),
            pl.BlockSpec((BT, TOP_K), lambda i: (i, 0)),
        ],
        out_shape=[
            jax.ShapeDtypeStruct((T, TOP_K), jnp.float32),
            jax.ShapeDtypeStruct((T, TOP_K), jnp.int32),
        ],
    )(x2, m2, wt, b2)
    return ew.reshape(B, S, TOP_K), ei.reshape(B, S, TOP_K)


# speedup vs baseline: 1.0983x; 1.0983x over previous
import functools

import jax
import jax.numpy as jnp
from jax.experimental import pallas as pl
from jax.experimental.pallas import tpu as pltpu

B, S, D, E, TOP_K = 4, 4096, 4096, 64, 8
BT = 1024


def _router_body(x_ref, m_ref, w_ref, b_ref, ew_ref, ei_ref):
    logits = jnp.dot(x_ref[...], w_ref[...], preferred_element_type=jnp.float32)
    logits = logits + b_ref[...]
    mask = m_ref[...]
    logits = jnp.where(mask != 1, -jnp.inf, logits)

    iota = jax.lax.broadcasted_iota(jnp.int32, (BT, E), 1)
    vals = logits
    top_vals = []
    top_idx = []
    for _ in range(TOP_K):
        m = jnp.max(vals, axis=-1, keepdims=True)
        idx = jnp.min(jnp.where(vals == m, iota, E), axis=-1, keepdims=True)
        top_vals.append(m)
        top_idx.append(idx)
        vals = jnp.where(iota == idx, -jnp.inf, vals)

    tv = jnp.concatenate(top_vals, axis=-1)
    ti = jnp.concatenate(top_idx, axis=-1)
    e = jnp.exp(tv - tv[:, 0:1])
    ew_ref[...] = e / jnp.sum(e, axis=-1, keepdims=True)
    ei_ref[...] = ti


@functools.partial(jax.jit, static_argnames=())
def kernel(x, attention_mask, W, b):
    T = B * S
    x2 = x.reshape(T, D)
    m2 = attention_mask.reshape(T, 1)
    wt = W.T
    b2 = b.reshape(1, E)

    grid = (T // BT,)
    ew, ei = pl.pallas_call(
        _router_body,
        grid=grid,
        in_specs=[
            pl.BlockSpec((BT, D), lambda i: (i, 0)),
            pl.BlockSpec((BT, 1), lambda i: (i, 0)),
            pl.BlockSpec((D, E), lambda i: (0, 0)),
            pl.BlockSpec((1, E), lambda i: (0, 0)),
        ],
        out_specs=[
            pl.BlockSpec((BT, TOP_K), lambda i: (i, 0)),
            pl.BlockSpec((BT, TOP_K), lambda i: (i, 0)),
        ],
        out_shape=[
            jax.ShapeDtypeStruct((T, TOP_K), jnp.float32),
            jax.ShapeDtypeStruct((T, TOP_K), jnp.int32),
        ],
    )(x2, m2, wt, b2)
    return ew.reshape(B, S, TOP_K), ei.reshape(B, S, TOP_K)


# speedup vs baseline: 1.3455x; 1.2251x over previous
import functools

import jax
import jax.numpy as jnp
from jax import lax
from jax.experimental import pallas as pl
from jax.experimental.pallas import tpu as pltpu
from jax.experimental.pallas import tpu_sc as plsc

B, S, D, E, TOP_K = 4, 4096, 4096, 64, 8
T = B * S

BT = 1024

NC, NS, L = 2, 16, 16
NW = NC * NS
TOK_W = T // NW
NG = TOK_W // L


def _logits_body(x_ref, m_ref, w_ref, b_ref, lg_ref):
    lg = lax.dot_general(
        w_ref[...], x_ref[...],
        dimension_numbers=(((1,), (1,)), ((), ())),
        preferred_element_type=jnp.float32,
    )
    lg = lg + b_ref[...]
    lg_ref[...] = jnp.where(m_ref[...] != 1, -jnp.inf, lg)


def _route_body(lg_hbm, ew_hbm, ei_hbm, lg_v, ew_v, ei_v, sem):
    wid = lax.axis_index("s") * NC + lax.axis_index("c")
    base = wid * TOK_W
    pltpu.sync_copy(lg_hbm.at[:, pl.ds(base, TOK_W)], lg_v)

    lane = lax.iota(jnp.int32, L)
    neg_inf = jnp.full((L,), -jnp.inf, jnp.float32)

    def group(g, _):
        topv = [neg_inf] * TOP_K
        topi = [jnp.zeros((L,), jnp.int32)] * TOP_K
        for e in range(E):
            xv = lg_v[e, pl.ds(g * L, L)]
            xi = jnp.full((L,), e, jnp.int32)
            for j in range(TOP_K):
                c = xv > topv[j]
                nv = jnp.where(c, xv, topv[j])
                xv = jnp.where(c, topv[j], xv)
                ni = jnp.where(c, xi, topi[j])
                xi = jnp.where(c, topi[j], xi)
                topv[j] = nv
                topi[j] = ni
        es = [jnp.exp(v - topv[0]) for v in topv]
        tot = es[0]
        for v in es[1:]:
            tot = tot + v
        for j in range(TOP_K):
            ew_v[j, pl.ds(g * L, L)] = es[j] / tot
            ei_v[j, pl.ds(g * L, L)] = topi[j]
        return 0

    lax.fori_loop(0, NG, group, 0)
    pltpu.sync_copy(ew_v, ew_hbm.at[:, pl.ds(base, TOK_W)])
    pltpu.sync_copy(ei_v, ei_hbm.at[:, pl.ds(base, TOK_W)])


_route = functools.partial(
    pl.kernel,
    out_type=[
        jax.ShapeDtypeStruct((TOP_K, T), jnp.float32),
        jax.ShapeDtypeStruct((TOP_K, T), jnp.int32),
    ],
    mesh=plsc.VectorSubcoreMesh(core_axis_name="c", subcore_axis_name="s"),
    scratch_types=[
        pltpu.VMEM((E, TOK_W), jnp.float32),
        pltpu.VMEM((TOP_K, TOK_W), jnp.float32),
        pltpu.VMEM((TOP_K, TOK_W), jnp.int32),
        pltpu.SemaphoreType.DMA,
    ],
)(_route_body)


@jax.jit
def kernel(x, attention_mask, W, b):
    x2 = x.reshape(T, D)
    m2 = attention_mask.reshape(1, T)
    b2 = b.reshape(E, 1)

    logits = pl.pallas_call(
        _logits_body,
        grid=(T // BT,),
        in_specs=[
            pl.BlockSpec((BT, D), lambda i: (i, 0)),
            pl.BlockSpec((1, BT), lambda i: (0, i)),
            pl.BlockSpec((E, D), lambda i: (0, 0)),
            pl.BlockSpec((E, 1), lambda i: (0, 0)),
        ],
        out_specs=pl.BlockSpec((E, BT), lambda i: (0, i)),
        out_shape=jax.ShapeDtypeStruct((E, T), jnp.float32),
    )(x2, m2, W, b2)

    ew, ei = _route(logits)
    return (
        ew.T.reshape(B, S, TOP_K),
        ei.T.reshape(B, S, TOP_K),
    )
